# Initial kernel scaffold; baseline (speedup 1.0000x reference)
import jax
import jax.numpy as jnp
from jax.experimental import pallas as pl


def kernel(x, edge_index, batch, W1, b1, g1, bt1, W2, b2, g2, bt2, W3, b3, g3, bt3, fW1, fb1, fW2, fb2, fW3, fb3):
    raise NotImplementedError("write your pallas kernel here")


# speedup vs baseline: 10.4339x; 10.4339x over previous
import functools

import jax
import jax.numpy as jnp
from jax import lax
from jax.experimental import pallas as pl
from jax.experimental.pallas import tpu as pltpu
from jax.experimental.pallas import tpu_sc as plsc

_EPS = 1e-5
_RB = 512
_K = 128
_NSUB = 16
_NW = 32


def _cdiv(a, b):
    return (a + b - 1) // b


def _sc_degree(dst, n_pad):
    e = dst.shape[0]
    nchunk = e // _K
    rps = n_pad // _NSUB
    mesh = plsc.VectorSubcoreMesh(core_axis_name="c", subcore_axis_name="s")

    def body(dst_hbm, zeros_hbm, ones_hbm, out_hbm, dst_v, rows_v, acc, sem):
        c = lax.axis_index("c")
        s = lax.axis_index("s")
        wid = c * _NSUB + s
        pltpu.async_copy(ones_hbm, rows_v, sem).wait()
        pltpu.sync_copy(zeros_hbm, acc.at[pl.ds(s * rps, rps)])
        plsc.subcore_barrier()
        nch = (nchunk - wid + _NW - 1) // _NW

        def chunk(i, carry):
            e0 = (wid + i * _NW) * _K
            pltpu.sync_copy(dst_hbm.at[pl.ds(e0, _K)], dst_v)
            pltpu.sync_copy(rows_v, acc.at[dst_v], add=True)
            return carry

        lax.fori_loop(0, nch, chunk, 0)
        plsc.subcore_barrier()
        pltpu.sync_copy(acc.at[pl.ds(s * rps, rps)],
                        out_hbm.at[c, pl.ds(s * rps, rps)])

    f = pl.kernel(
        body,
        out_type=jax.ShapeDtypeStruct((2, n_pad, 16), jnp.float32),
        mesh=mesh,
        compiler_params=pltpu.CompilerParams(use_tc_tiling_on_sc=False),
        scratch_types=[
            pltpu.VMEM((_K,), jnp.int32),
            pltpu.VMEM((_K, 16), jnp.float32),
            pltpu.VMEM_SHARED((n_pad, 16), jnp.float32),
            pltpu.SemaphoreType.DMA,
        ],
    )
    zeros = jnp.zeros((rps, 16), jnp.float32)
    ones = jnp.ones((_K, 16), jnp.float32)
    return f(dst, zeros, ones)


def _sc_aggregate(src, dst, tables, n_pad, width):
    e = src.shape[0]
    nchunk = e // _K
    nt = len(tables)
    rps = n_pad // _NSUB
    mesh = plsc.VectorSubcoreMesh(core_axis_name="c", subcore_axis_name="s")

    def body(src_hbm, dst_hbm, zeros_hbm, *rest):
        tabs = rest[:nt]
        out_hbm = rest[nt]
        src_v, dst_v, rows_v, acc, sem = rest[nt + 1:]
        c = lax.axis_index("c")
        s = lax.axis_index("s")
        wid = c * _NSUB + s
        nch = (nchunk - wid + _NW - 1) // _NW
        for t in range(nt):
            pltpu.sync_copy(zeros_hbm, acc.at[pl.ds(s * rps, rps)])
            plsc.subcore_barrier()

            def chunk(i, carry, t=t):
                e0 = (wid + i * _NW) * _K
                pltpu.sync_copy(src_hbm.at[pl.ds(e0, _K)], src_v)
                pltpu.sync_copy(dst_hbm.at[pl.ds(e0, _K)], dst_v)
                pltpu.async_copy(tabs[t].at[src_v], rows_v, sem).wait()
                pltpu.sync_copy(rows_v, acc.at[dst_v], add=True)
                return carry

            lax.fori_loop(0, nch, chunk, 0)
            plsc.subcore_barrier()
            pltpu.sync_copy(acc.at[pl.ds(s * rps, rps)],
                            out_hbm.at[c, t, pl.ds(s * rps, rps)])
            plsc.subcore_barrier()

    f = pl.kernel(
        body,
        out_type=jax.ShapeDtypeStruct((2, nt, n_pad, width), jnp.float32),
        mesh=mesh,
        compiler_params=pltpu.CompilerParams(use_tc_tiling_on_sc=False),
        scratch_types=[
            pltpu.VMEM((_K,), jnp.int32),
            pltpu.VMEM((_K,), jnp.int32),
            pltpu.VMEM((_K, width), jnp.float32),
            pltpu.VMEM_SHARED((n_pad, width), jnp.float32),
            pltpu.SemaphoreType.DMA,
        ],
    )
    zeros = jnp.zeros((rps, width), jnp.float32)
    return f(src, dst, zeros, *tables)


def _tc_prep(xpad, deg_s):
    n_pad = xpad.shape[0]
    nblk = n_pad // _RB

    def body(x_ref, d_ref, p1_ref, dv_ref):
        deg = d_ref[0, :, 0:1] + d_ref[1, :, 0:1] + 1.0
        dv = lax.rsqrt(deg)
        dv_ref[...] = jnp.broadcast_to(dv, (_RB, 16))
        p1_ref[...] = x_ref[...] * dv

    return pl.pallas_call(
        body,
        grid=(nblk,),
        in_specs=[
            pl.BlockSpec((_RB, 16), lambda i: (i, 0)),
            pl.BlockSpec((2, _RB, 16), lambda i: (0, i, 0)),
        ],
        out_specs=[
            pl.BlockSpec((_RB, 16), lambda i: (i, 0)),
            pl.BlockSpec((_RB, 16), lambda i: (i, 0)),
        ],
        out_shape=[
            jax.ShapeDtypeStruct((n_pad, 16), jnp.float32),
            jax.ShapeDtypeStruct((n_pad, 16), jnp.float32),
        ],
    )(xpad, deg_s)


def _tc_combine(S, ps, dinv, W, b, n_real):
    n_pad = ps[0].shape[0]
    nt = len(ps)
    w = ps[0].shape[1]
    fo = W.shape[1] if W is not None else nt * w
    nblk = n_pad // _RB

    def body(s_ref, *refs):
        p_refs = refs[:nt]
        dv_ref = refs[nt]
        idx = nt + 1
        if W is not None:
            w_ref = refs[idx]
            idx += 1
        b_ref = refs[idx]
        u_ref, st_ref, acc = refs[idx + 1], refs[idx + 2], refs[idx + 3]
        i = pl.program_id(0)
        dv = dv_ref[:, 0:1]
        aggs = [(s_ref[0, t] + s_ref[1, t] + p_refs[t][...]) * dv
                for t in range(nt)]
        if W is not None:
            u = sum(jnp.dot(aggs[t], w_ref[t * w:(t + 1) * w, :],
                            preferred_element_type=jnp.float32)
                    for t in range(nt))
        else:
            u = jnp.concatenate(aggs, axis=1)
        u = u + b_ref[...]
        rows = lax.broadcasted_iota(jnp.int32, (_RB, 1), 0) + i * _RB
        u = jnp.where(rows < n_real, u, 0.0)
        u_ref[...] = u

        @pl.when(i == 0)
        def _():
            acc[...] = jnp.zeros_like(acc)

        acc[0:1, :] = acc[0:1, :] + jnp.sum(u, axis=0, keepdims=True)
        acc[1:2, :] = acc[1:2, :] + jnp.sum(u * u, axis=0, keepdims=True)

        @pl.when(i == nblk - 1)
        def _():
            st_ref[...] = acc[...]

    in_specs = [pl.BlockSpec((2, nt, _RB, w), lambda i: (0, 0, i, 0))]
    in_specs += [pl.BlockSpec((_RB, w), lambda i: (i, 0))] * nt
    in_specs.append(pl.BlockSpec((_RB, 16), lambda i: (i, 0)))
    args = [S] + list(ps) + [dinv]
    if W is not None:
        in_specs.append(pl.BlockSpec(W.shape, lambda i: (0, 0)))
        args.append(W)
    in_specs.append(pl.BlockSpec((1, fo), lambda i: (0, 0)))
    args.append(b.reshape(1, fo))
    return pl.pallas_call(
        body,
        grid=(nblk,),
        in_specs=in_specs,
        out_specs=[
            pl.BlockSpec((_RB, fo), lambda i: (i, 0)),
            pl.BlockSpec((8, fo), lambda i: (0, 0)),
        ],
        out_shape=[
            jax.ShapeDtypeStruct((n_pad, fo), jnp.float32),
            jax.ShapeDtypeStruct((8, fo), jnp.float32),
        ],
        scratch_shapes=[pltpu.VMEM((8, fo), jnp.float32)],
    )(*args)


def _tc_scale_next(u, stats, g, bt, dinv, w_next, n_real):
    n_pad, f = u.shape
    fo = w_next.shape[1] if w_next is not None else f
    nblk = n_pad // _RB
    inv_n = 1.0 / n_real

    def body(u_ref, st_ref, g_ref, bt_ref, dv_ref, *rest):
        if w_next is not None:
            w_ref = rest[0]
            rest = rest[1:]
        oa_ref, ob_ref = rest
        mean = st_ref[0:1, :] * inv_n
        var = st_ref[1:2, :] * inv_n - mean * mean
        sc = g_ref[...] * lax.rsqrt(var + _EPS)
        sh = bt_ref[...] - mean * sc
        y = jnp.maximum(u_ref[...] * sc + sh, 0.0)
        if w_next is not None:
            y = jnp.dot(y, w_ref[...], preferred_element_type=jnp.float32)
        p = y * dv_ref[:, 0:1]
        oa_ref[...] = p[:, :fo // 2]
        ob_ref[...] = p[:, fo // 2:]

    in_specs = [
        pl.BlockSpec((_RB, f), lambda i: (i, 0)),
        pl.BlockSpec((8, f), lambda i: (0, 0)),
        pl.BlockSpec((1, f), lambda i: (0, 0)),
        pl.BlockSpec((1, f), lambda i: (0, 0)),
        pl.BlockSpec((_RB, 16), lambda i: (i, 0)),
    ]
    args = [u, stats, g.reshape(1, f), bt.reshape(1, f), dinv]
    if w_next is not None:
        in_specs.append(pl.BlockSpec(w_next.shape, lambda i: (0, 0)))
        args.append(w_next)
    return pl.pallas_call(
        body,
        grid=(nblk,),
        in_specs=in_specs,
        out_specs=[
            pl.BlockSpec((_RB, fo // 2), lambda i: (i, 0)),
            pl.BlockSpec((_RB, fo // 2), lambda i: (i, 0)),
        ],
        out_shape=[
            jax.ShapeDtypeStruct((n_pad, fo // 2), jnp.float32),
            jax.ShapeDtypeStruct((n_pad, fo // 2), jnp.float32),
        ],
    )(*args)


def _tc_pool(u, stats, g, bt, ids3, n_real, nb):
    n_pad, f = u.shape
    nblk = n_pad // _RB
    inv_n = 1.0 / n_real

    def body(u_ref, st_ref, g_ref, bt_ref, idv_ref, ids_ref,
             sums_ref, cnts_ref, maxs_ref, sacc, cacc, macc):
        i = pl.program_id(0)

        @pl.when(i == 0)
        def _():
            sacc[...] = jnp.zeros_like(sacc)
            cacc[...] = jnp.zeros_like(cacc)
            macc[...] = jnp.full_like(macc, -jnp.inf)

        mean = st_ref[0:1, :] * inv_n
        var = st_ref[1:2, :] * inv_n - mean * mean
        sc = g_ref[...] * lax.rsqrt(var + _EPS)
        sh = bt_ref[...] - mean * sc
        y = jnp.maximum(u_ref[...] * sc + sh, 0.0)

        ids_r = idv_ref[0]
        seg = lax.broadcasted_iota(jnp.int32, (nb, _RB), 0)
        m = (seg == ids_r).astype(jnp.float32)
        sacc[...] = sacc[...] + jnp.dot(m, y, preferred_element_type=jnp.float32)
        cacc[...] = cacc[...] + jnp.sum(m, axis=1, keepdims=True)

        first = ids_ref[0, 0, 0]
        last = ids_ref[0, 0, _RB - 1]
        ids_col = ids_r.reshape(_RB, 1)

        def mbody(sv, carry):
            msk = ids_col == sv
            contrib = jnp.max(jnp.where(msk, y, -jnp.inf), axis=0,
                              keepdims=True)
            macc[pl.ds(sv, 1), :] = jnp.maximum(macc[pl.ds(sv, 1), :], contrib)
            return carry

        lax.fori_loop(first, last + 1, mbody, 0)

        @pl.when(i == nblk - 1)
        def _():
            sums_ref[...] = sacc[...]
            cnts_ref[...] = cacc[...]
            maxs_ref[...] = macc[0:nb, :]

    return pl.pallas_call(
        body,
        grid=(nblk,),
        in_specs=[
            pl.BlockSpec((_RB, f), lambda i: (i, 0)),
            pl.BlockSpec((8, f), lambda i: (0, 0)),
            pl.BlockSpec((1, f), lambda i: (0, 0)),
            pl.BlockSpec((1, f), lambda i: (0, 0)),
            pl.BlockSpec((1, 1, _RB), lambda i: (i, 0, 0)),
            pl.BlockSpec((1, 1, _RB), lambda i: (i, 0, 0),
                         memory_space=pltpu.SMEM),
        ],
        out_specs=[
            pl.BlockSpec((nb, f), lambda i: (0, 0)),
            pl.BlockSpec((nb, 1), lambda i: (0, 0)),
            pl.BlockSpec((nb, f), lambda i: (0, 0)),
        ],
        out_shape=[
            jax.ShapeDtypeStruct((nb, f), jnp.float32),
            jax.ShapeDtypeStruct((nb, 1), jnp.float32),
            jax.ShapeDtypeStruct((nb, f), jnp.float32),
        ],
        scratch_shapes=[
            pltpu.VMEM((nb, f), jnp.float32),
            pltpu.VMEM((nb, 1), jnp.float32),
            pltpu.VMEM((nb + 8, f), jnp.float32),
        ],
    )(u, stats, g.reshape(1, f), bt.reshape(1, f), ids3, ids3)


def _tc_mlp(sums, cnts, maxs, fw1, fb1, fw2, fb2, fw3, fb3):
    nb, f = sums.shape

    def body(s_ref, c_ref, m_ref, w1_ref, b1_ref, w2_ref, b2_ref,
             w3_ref, b3_ref, o_ref):
        mean = s_ref[...] / jnp.maximum(c_ref[:, 0:1], 1.0)
        z = (jnp.dot(mean, w1_ref[0:f, :], preferred_element_type=jnp.float32)
             + jnp.dot(m_ref[...], w1_ref[f:2 * f, :],
                       preferred_element_type=jnp.float32)
             + b1_ref[...])
        z = jnp.maximum(z, 0.0)
        z = jnp.dot(z, w2_ref[...], preferred_element_type=jnp.float32) + b2_ref[...]
        z = jnp.maximum(z, 0.0)
        o_ref[...] = (jnp.dot(z, w3_ref[...], preferred_element_type=jnp.float32)
                      + b3_ref[...])

    return pl.pallas_call(
        body,
        out_shape=jax.ShapeDtypeStruct((nb, 1), jnp.float32),
    )(sums, cnts, maxs, fw1, fb1.reshape(1, -1), fw2, fb2.reshape(1, -1),
      fw3, fb3.reshape(1, -1))


def kernel(x, edge_index, batch, W1, b1, g1, bt1, W2, b2, g2, bt2,
           W3, b3, g3, bt3, fW1, fb1, fW2, fb2, fW3, fb3):
    n, fin = x.shape
    nb = 512
    n_pad = _RB * _cdiv(n, _RB)

    src = edge_index[0]
    dst = edge_index[1]
    xpad = jnp.pad(x, ((0, n_pad - n), (0, 16 - fin)))
    w1p = jnp.pad(W1, ((0, 16 - fin), (0, 0)))

    deg_s = _sc_degree(dst, n_pad)
    p1, dinv = _tc_prep(xpad, deg_s)

    s1 = _sc_aggregate(src, dst, [p1], n_pad, 16)
    u1, st1 = _tc_combine(s1, [p1], dinv, w1p, b1, n)
    p2a, p2b = _tc_scale_next(u1, st1, g1, bt1, dinv, None, n)

    s2 = _sc_aggregate(src, dst, [p2a, p2b], n_pad, 32)
    u2, st2 = _tc_combine(s2, [p2a, p2b], dinv, W2, b2, n)
    p3a, p3b = _tc_scale_next(u2, st2, g2, bt2, dinv, W3, n)

    s3 = _sc_aggregate(src, dst, [p3a, p3b], n_pad, 32)
    u3, st3 = _tc_combine(s3, [p3a, p3b], dinv, None, b3, n)

    ids3 = jnp.pad(batch, (0, n_pad - n), constant_values=nb)
    ids3 = ids3.reshape(n_pad // _RB, 1, _RB)
    sums, cnts, maxs = _tc_pool(u3, st3, g3, bt3, ids3, n, nb)

    return _tc_mlp(sums, cnts, maxs, fW1, fb1, fW2, fb2, fW3, fb3)
